# baseline (speedup 1.0000x reference)
import functools

import jax
import jax.numpy as jnp
from jax import lax
from jax.experimental import pallas as pl
from jax.experimental.pallas import tpu as pltpu
from jax.experimental.pallas import tpu_sc as plsc

N = 10000
NPAD = 10240
E = 320000
NNEG = 50000
NNEG_PAD = 51200
F_IN = 128
H = 64
NC = 2
NS = 16
NW = NC * NS
RPT = NPAD // NS
FH = H // NC

CHUNK_D = 2000
CK = 400
NCH_P = E // CK
NCH_N = NNEG_PAD // CK
CPT_P = NCH_P // NS
CPT_N = NCH_N // NS
ETOT = E + NNEG_PAD

_mesh = plsc.VectorSubcoreMesh(core_axis_name="c", subcore_axis_name="s")
_sc_params = pltpu.CompilerParams(use_tc_tiling_on_sc=False)
_sc_params_nl = pltpu.CompilerParams(use_tc_tiling_on_sc=False,
                                     needs_layout_passes=False)


def _sc_degree_body(col_hbm, out_hbm, deg_sp, cidx_v, ones_v):
    cid = lax.axis_index("c")
    sid = lax.axis_index("s")

    def zfill(i, _):
        ones_v[pl.ds(i * 16, 16)] = jnp.zeros((16,), jnp.float32)
        return 0

    lax.fori_loop(0, RPT // 16, zfill, 0)
    pltpu.sync_copy(ones_v.at[pl.ds(0, RPT)], deg_sp.at[pl.ds(sid * RPT, RPT)])

    def fill(i, _):
        ones_v[pl.ds(i * 16, 16)] = jnp.ones((16,), jnp.float32)
        return 0

    lax.fori_loop(0, CHUNK_D // 16, fill, 0)
    plsc.subcore_barrier()

    epw = E // NW
    base = (cid * NS + sid) * epw

    def body(k, _):
        pltpu.sync_copy(col_hbm.at[pl.ds(base + k * CHUNK_D, CHUNK_D)], cidx_v)
        pltpu.sync_copy(ones_v, deg_sp.at[cidx_v], add=True)
        return 0

    lax.fori_loop(0, epw // CHUNK_D, body, 0)
    plsc.subcore_barrier()
    pltpu.sync_copy(deg_sp.at[pl.ds(sid * RPT, RPT)],
                    out_hbm.at[cid, pl.ds(sid * RPT, RPT)])


def _sc_degree(col):
    k = functools.partial(
        pl.kernel,
        out_type=jax.ShapeDtypeStruct((NC, NPAD), jnp.float32),
        mesh=_mesh,
        compiler_params=_sc_params,
        scratch_types=[
            pltpu.VMEM_SHARED((NPAD,), jnp.float32),
            pltpu.VMEM((CHUNK_D,), jnp.int32),
            pltpu.VMEM((CHUNK_D,), jnp.float32),
        ],
    )(_sc_degree_body)
    return k(col)


def _edge_ring_loop(y_sp, acc_sp, ridx_all, cidx_all, bufs, gsems, ssems):
    def g_issue(k, i):
        pltpu.async_copy(y_sp.at[ridx_all.at[k]], bufs[i], gsems[i])

    def g_wait(k, i):
        pltpu.make_async_copy(y_sp.at[ridx_all.at[k]], bufs[i],
                              gsems[i]).wait()

    def s_issue(k, i):
        pltpu.async_copy(bufs[i], acc_sp.at[cidx_all.at[k]], ssems[i],
                         add=True)

    def s_wait(i):
        pltpu.make_async_copy(bufs[i], acc_sp.at[cidx_all.at[0]],
                              ssems[i]).wait()

    g_issue(0, 0)
    g_issue(1, 1)
    g_issue(2, 2)

    def body(u, _):
        k = 3 * u
        g_wait(k, 0)
        s_issue(k, 0)
        g_wait(k + 1, 1)
        s_issue(k + 1, 1)
        g_wait(k + 2, 2)
        s_issue(k + 2, 2)
        for i in range(3):
            knext = k + 3 + i

            @pl.when(knext < CPT_P)
            def _():
                s_wait(i)
                g_issue(knext, i)

        return 0

    lax.fori_loop(0, CPT_P // 3, body, 0)
    g_wait(CPT_P - 2, 0)
    s_issue(CPT_P - 2, 0)
    g_wait(CPT_P - 1, 1)
    s_issue(CPT_P - 1, 1)
    s_wait(2)
    s_wait(0)
    s_wait(1)


def _sc_edge_body(y_hbm, row2d_hbm, col2d_hbm, out_hbm,
                  y_sp, acc_sp, ridx_all, cidx_all,
                  rows_a, rows_b, rows_c,
                  sem_g0, sem_g1, sem_g2, sem_s0, sem_s1, sem_s2):
    cid = lax.axis_index("c")
    sid = lax.axis_index("s")
    fbase = cid * FH
    r0 = sid * RPT

    def zfill(i, _):
        rows_a[i // 2, pl.ds((i % 2) * 16, 16)] = jnp.zeros((16,), jnp.float32)
        return 0

    lax.fori_loop(0, CK * 2, zfill, 0)
    pltpu.sync_copy(rows_a, acc_sp.at[pl.ds(r0, CK)])
    pltpu.sync_copy(rows_a.at[pl.ds(0, RPT - CK)],
                    acc_sp.at[pl.ds(r0 + CK, RPT - CK)])
    pltpu.sync_copy(y_hbm.at[pl.ds(r0, RPT), pl.ds(fbase, FH)],
                    y_sp.at[pl.ds(r0, RPT)])
    pltpu.sync_copy(row2d_hbm.at[pl.ds(sid * CPT_P, CPT_P)], ridx_all)
    pltpu.sync_copy(col2d_hbm.at[pl.ds(sid * CPT_P, CPT_P)], cidx_all)
    plsc.subcore_barrier()
    _edge_ring_loop(y_sp, acc_sp, ridx_all, cidx_all,
                    (rows_a, rows_b, rows_c),
                    (sem_g0, sem_g1, sem_g2), (sem_s0, sem_s1, sem_s2))
    plsc.subcore_barrier()
    pltpu.sync_copy(acc_sp.at[pl.ds(r0, RPT)],
                    out_hbm.at[pl.ds(r0, RPT), pl.ds(fbase, FH)])


def _sc_edge_pass(y_pad, row2d, col2d):
    k = functools.partial(
        pl.kernel,
        out_type=jax.ShapeDtypeStruct((NPAD, H), jnp.float32),
        mesh=_mesh,
        compiler_params=_sc_params,
        scratch_types=[
            pltpu.VMEM_SHARED((NPAD, FH), jnp.float32),
            pltpu.VMEM_SHARED((NPAD, FH), jnp.float32),
            pltpu.VMEM((CPT_P, CK), jnp.int32),
            pltpu.VMEM((CPT_P, CK), jnp.int32),
            pltpu.VMEM((CK, FH), jnp.float32),
            pltpu.VMEM((CK, FH), jnp.float32),
            pltpu.VMEM((CK, FH), jnp.float32),
            pltpu.SemaphoreType.DMA,
            pltpu.SemaphoreType.DMA,
            pltpu.SemaphoreType.DMA,
            pltpu.SemaphoreType.DMA,
            pltpu.SemaphoreType.DMA,
            pltpu.SemaphoreType.DMA,
        ],
    )(_sc_edge_body)
    return k(y_pad, row2d, col2d)


def _sc_edge_final_body(y_hbm, row2d_hbm, col2d_hbm, dinv_hbm, b2_hbm,
                        rep_out,
                        y_sp, acc_sp, ridx_all, cidx_all,
                        rows_a, rows_b, rows_c, dinv_v, b2h,
                        sem_g0, sem_g1, sem_g2, sem_s0, sem_s1, sem_s2):
    cid = lax.axis_index("c")
    sid = lax.axis_index("s")
    fbase = cid * FH
    r0 = sid * RPT

    def zfill(i, _):
        rows_a[i // 2, pl.ds((i % 2) * 16, 16)] = jnp.zeros((16,), jnp.float32)
        return 0

    lax.fori_loop(0, CK * 2, zfill, 0)
    pltpu.sync_copy(rows_a, acc_sp.at[pl.ds(r0, CK)])
    pltpu.sync_copy(rows_a.at[pl.ds(0, RPT - CK)],
                    acc_sp.at[pl.ds(r0 + CK, RPT - CK)])
    pltpu.sync_copy(y_hbm.at[pl.ds(r0, RPT), pl.ds(fbase, FH)],
                    y_sp.at[pl.ds(r0, RPT)])
    pltpu.sync_copy(row2d_hbm.at[pl.ds(sid * CPT_P, CPT_P)], ridx_all)
    pltpu.sync_copy(col2d_hbm.at[pl.ds(sid * CPT_P, CPT_P)], cidx_all)
    pltpu.sync_copy(dinv_hbm.at[pl.ds(r0, RPT)], dinv_v.at[pl.ds(0, RPT)])
    pltpu.sync_copy(b2_hbm.at[pl.ds(fbase, FH)], b2h)
    plsc.subcore_barrier()
    _edge_ring_loop(y_sp, acc_sp, ridx_all, cidx_all,
                    (rows_a, rows_b, rows_c),
                    (sem_g0, sem_g1, sem_g2), (sem_s0, sem_s1, sem_s2))
    plsc.subcore_barrier()

    b2v0 = b2h[pl.ds(0, 16)]
    b2v1 = b2h[pl.ds(16, 16)]

    def span(off, length):
        pltpu.sync_copy(acc_sp.at[pl.ds(r0 + off, length)],
                        rows_a.at[pl.ds(0, length)])
        pltpu.sync_copy(y_sp.at[pl.ds(r0 + off, length)],
                        rows_b.at[pl.ds(0, length)])

        def rw(r, _):
            dv = dinv_v[pl.ds(off + r, 16)][0]
            v0 = rows_a[r, pl.ds(0, 16)] + rows_b[r, pl.ds(0, 16)]
            v1 = rows_a[r, pl.ds(16, 16)] + rows_b[r, pl.ds(16, 16)]
            rows_a[r, pl.ds(0, 16)] = v0 * dv + b2v0
            rows_a[r, pl.ds(16, 16)] = v1 * dv + b2v1
            return 0

        lax.fori_loop(0, length, rw, 0)
        pltpu.sync_copy(rows_a.at[pl.ds(0, length)],
                        rep_out.at[pl.ds(r0 + off, length), pl.ds(fbase, FH)])

    span(0, CK)

    @pl.when(sid < NS - 1)
    def _():
        span(CK, RPT - CK)


def _sc_edge_final(y_pad, row2d, col2d, dinv1d, b2):
    k = functools.partial(
        pl.kernel,
        out_type=jax.ShapeDtypeStruct((N, H), jnp.float32),
        mesh=_mesh,
        compiler_params=_sc_params,
        scratch_types=[
            pltpu.VMEM_SHARED((NPAD, FH), jnp.float32),
            pltpu.VMEM_SHARED((NPAD, FH), jnp.float32),
            pltpu.VMEM((CPT_P, CK), jnp.int32),
            pltpu.VMEM((CPT_P, CK), jnp.int32),
            pltpu.VMEM((CK, FH), jnp.float32),
            pltpu.VMEM((CK, FH), jnp.float32),
            pltpu.VMEM((CK, FH), jnp.float32),
            pltpu.VMEM((RPT + 16,), jnp.float32),
            pltpu.VMEM((FH,), jnp.float32),
            pltpu.SemaphoreType.DMA,
            pltpu.SemaphoreType.DMA,
            pltpu.SemaphoreType.DMA,
            pltpu.SemaphoreType.DMA,
            pltpu.SemaphoreType.DMA,
            pltpu.SemaphoreType.DMA,
        ],
    )(_sc_edge_final_body)
    return k(y_pad, row2d, col2d, dinv1d, b2)


def _sc_score_body(rep_hbm, pr_hbm, pc_hbm, nr_hbm, nc_hbm, out_hbm,
                   rep_sp, pr_idx, pc_idx, nr_idx, nc_idx,
                   ar0, br0, ar1, br1, d0, d1,
                   sem_g0, sem_g1, sem_w0, sem_w1):
    cid = lax.axis_index("c")
    sid = lax.axis_index("s")
    fbase = cid * FH
    r0 = sid * RPT

    rs0 = sid * (N // NS)
    pltpu.sync_copy(rep_hbm.at[pl.ds(rs0, N // NS), pl.ds(fbase, FH)],
                    rep_sp.at[pl.ds(rs0, N // NS)])
    pltpu.sync_copy(pr_hbm.at[pl.ds(sid * CPT_P, CPT_P)], pr_idx)
    pltpu.sync_copy(pc_hbm.at[pl.ds(sid * CPT_P, CPT_P)], pc_idx)
    pltpu.sync_copy(nr_hbm.at[pl.ds(sid * CPT_N, CPT_N)], nr_idx)
    pltpu.sync_copy(nc_hbm.at[pl.ds(sid * CPT_N, CPT_N)], nc_idx)
    plsc.subcore_barrier()

    lane = lax.iota(jnp.int32, 16)

    def compute(arows, brows, dbuf):
        def grp(g, _):
            rows0 = g * 16 + lane
            accs = [jnp.zeros((16,), jnp.float32) for _ in range(4)]
            for f in range(FH):
                colsf = jnp.bitwise_and(f + lane, FH - 1)
                a = plsc.load_gather(arows, [rows0, colsf])
                b = plsc.load_gather(brows, [rows0, colsf])
                accs[f % 4] = accs[f % 4] + a * b
            dbuf[pl.ds(g * 16, 16)] = (accs[0] + accs[1]) + (accs[2] + accs[3])
            return 0

        lax.fori_loop(0, CK // 16, grp, 0)

    def run(cpt, ridx, cidx, obase):
        def gather(t, ar, br, sem):
            pltpu.async_copy(rep_sp.at[ridx.at[t]], ar, sem)
            pltpu.async_copy(rep_sp.at[cidx.at[t]], br, sem)

        def drain(t, ar, br, sem):
            pltpu.make_async_copy(rep_sp.at[ridx.at[t]], ar, sem).wait()
            pltpu.make_async_copy(rep_sp.at[cidx.at[t]], br, sem).wait()

        gather(0, ar0, br0, sem_g0)

        def body(j, _):
            k0 = 2 * j
            drain(k0, ar0, br0, sem_g0)
            gather(k0 + 1, ar1, br1, sem_g1)

            @pl.when(j > 0)
            def _():
                pltpu.make_async_copy(
                    d0, out_hbm.at[cid, pl.ds(0, CK)], sem_w0).wait()

            compute(ar0, br0, d0)
            off0 = obase + (sid * cpt + k0) * CK
            pltpu.async_copy(d0, out_hbm.at[cid, pl.ds(off0, CK)], sem_w0)

            drain(k0 + 1, ar1, br1, sem_g1)

            @pl.when(k0 + 2 < cpt)
            def _():
                gather(k0 + 2, ar0, br0, sem_g0)

            @pl.when(j > 0)
            def _():
                pltpu.make_async_copy(
                    d1, out_hbm.at[cid, pl.ds(0, CK)], sem_w1).wait()

            compute(ar1, br1, d1)
            off1 = off0 + CK
            pltpu.async_copy(d1, out_hbm.at[cid, pl.ds(off1, CK)], sem_w1)
            return 0

        lax.fori_loop(0, cpt // 2, body, 0)
        pltpu.make_async_copy(d0, out_hbm.at[cid, pl.ds(0, CK)], sem_w0).wait()
        pltpu.make_async_copy(d1, out_hbm.at[cid, pl.ds(0, CK)], sem_w1).wait()

    run(CPT_P, pr_idx, pc_idx, 0)
    run(CPT_N, nr_idx, nc_idx, E)


def _sc_score(rep_pad, pr2d, pc2d, nr2d, nc2d):
    k = functools.partial(
        pl.kernel,
        out_type=jax.ShapeDtypeStruct((NC, ETOT), jnp.float32),
        mesh=_mesh,
        compiler_params=_sc_params_nl,
        scratch_types=[
            pltpu.VMEM_SHARED((N, FH), jnp.float32),
            pltpu.VMEM((CPT_P, CK), jnp.int32),
            pltpu.VMEM((CPT_P, CK), jnp.int32),
            pltpu.VMEM((CPT_N, CK), jnp.int32),
            pltpu.VMEM((CPT_N, CK), jnp.int32),
            pltpu.VMEM((CK, FH), jnp.float32),
            pltpu.VMEM((CK, FH), jnp.float32),
            pltpu.VMEM((CK, FH), jnp.float32),
            pltpu.VMEM((CK, FH), jnp.float32),
            pltpu.VMEM((CK,), jnp.float32),
            pltpu.VMEM((CK,), jnp.float32),
            pltpu.SemaphoreType.DMA,
            pltpu.SemaphoreType.DMA,
            pltpu.SemaphoreType.DMA,
            pltpu.SemaphoreType.DMA,
        ],
    )(_sc_score_body)
    return k(rep_pad, pr2d, pc2d, nr2d, nc2d)


def _prep_body(x_ref, w_ref, degt_ref, dinv_ref, y_ref):
    d = degt_ref[...]
    deg = d[:, 0:1] + d[:, 1:2] + 1.0
    dinv = lax.rsqrt(deg)
    dinv_ref[...] = dinv
    xw = jnp.dot(x_ref[...], w_ref[...], preferred_element_type=jnp.float32)
    y_ref[0:N, :] = dinv[0:N] * xw
    y_ref[N:NPAD, :] = jnp.zeros((NPAD - N, H), jnp.float32)


def _tc_prep(features, W1, degt):
    return pl.pallas_call(
        _prep_body,
        out_shape=[jax.ShapeDtypeStruct((NPAD, 1), jnp.float32),
                   jax.ShapeDtypeStruct((NPAD, H), jnp.float32)],
    )(features, W1, degt)


def _mid_body(acc_ref, y_ref, dinv_ref, b1_ref, w2_ref, o_ref):
    dinv = dinv_ref[...]
    s = acc_ref[...] + y_ref[...]
    h = jnp.maximum(dinv * s + b1_ref[...], 0.0)
    xw2 = jnp.dot(h, w2_ref[...], preferred_element_type=jnp.float32)
    y2 = dinv * xw2
    o_ref[0:N, :] = y2[0:N]
    o_ref[N:NPAD, :] = jnp.zeros((NPAD - N, H), jnp.float32)


def _tc_mid(acc1, y1p, dinvp, b1, W2):
    return pl.pallas_call(
        _mid_body,
        out_shape=jax.ShapeDtypeStruct((NPAD, H), jnp.float32),
    )(acc1, y1p, dinvp, b1, W2)


def _combine_body(dots_ref, pr_ref, pc_ref, nr_ref, nc_ref, o_ref):
    dp = dots_ref[0] + dots_ref[1]
    pos_d = dp[0:E // 128]
    neg_d = dp[E // 128:ETOT // 128]
    mp = (pr_ref[...] < pc_ref[...]).astype(jnp.float32)
    mn = (nr_ref[...] < nc_ref[...]).astype(jnp.float32)
    t = pos_d - 1.0
    s_pos = jnp.sum(mp * t * t)
    s_neg = jnp.sum(mn * neg_d * neg_d)
    denom = jnp.sum(mp) + jnp.sum(mn)
    rec = (s_neg + s_pos) * jnp.float32(N) / denom
    o_ref[...] = jnp.broadcast_to(rec, (1, 1))


def _tc_combine(dots3d, pr, pc, nr, nc):
    return pl.pallas_call(
        _combine_body,
        out_shape=jax.ShapeDtypeStruct((1, 1), jnp.float32),
    )(dots3d, pr, pc, nr, nc)


def kernel(features, edge_index, neg_edge_index, W1, b1, W2, b2):
    assert features.shape == (N, F_IN)
    assert edge_index.shape == (2, E)
    assert neg_edge_index.shape == (2, NNEG)

    row = edge_index[0]
    col = edge_index[1]
    nr = neg_edge_index[0]
    nc = neg_edge_index[1]
    zpad = jnp.zeros((NNEG_PAD - NNEG,), jnp.int32)
    nr_p = jnp.concatenate([nr, zpad])
    nc_p = jnp.concatenate([nc, zpad])
    row2d = row.reshape(NCH_P, CK)
    col2d = col.reshape(NCH_P, CK)
    nr2d = nr_p.reshape(NCH_N, CK)
    nc2d = nc_p.reshape(NCH_N, CK)

    degp = _sc_degree(col)
    degt = jnp.transpose(degp)
    dinvp, y1p = _tc_prep(features, W1, degt)
    acc1 = _sc_edge_pass(y1p, row2d, col2d)
    y2p = _tc_mid(acc1, y1p, dinvp, b1, W2)
    rep = _sc_edge_final(y2p, row2d, col2d, dinvp.reshape(NPAD), b2)
    dots = _sc_score(rep, row2d, col2d, nr2d, nc2d)
    dots3d = dots.reshape(NC, ETOT // 128, 128)
    rec_loss = _tc_combine(dots3d,
                           row.reshape(E // 128, 128),
                           col.reshape(E // 128, 128),
                           nr_p.reshape(NNEG_PAD // 128, 128),
                           nc_p.reshape(NNEG_PAD // 128, 128))[0, 0]
    return (rep, rec_loss)


# speedup vs baseline: 1.1426x; 1.1426x over previous
import functools

import jax
import jax.numpy as jnp
from jax import lax
from jax.experimental import pallas as pl
from jax.experimental.pallas import tpu as pltpu
from jax.experimental.pallas import tpu_sc as plsc

N = 10000
NPAD = 10240
E = 320000
NNEG = 50000
NNEG_PAD = 51200
F_IN = 128
H = 64
NC = 2
NS = 16
NW = NC * NS
RPT = NPAD // NS
FH = H // NC

CHUNK_D = 2000
CK = 400
NCH_P = E // CK
NCH_N = NNEG_PAD // CK
CPT_P = NCH_P // NS
CPT_N = NCH_N // NS
ETOT = E + NNEG_PAD

_mesh = plsc.VectorSubcoreMesh(core_axis_name="c", subcore_axis_name="s")
_sc_params = pltpu.CompilerParams(use_tc_tiling_on_sc=False)
_sc_params_nl = pltpu.CompilerParams(use_tc_tiling_on_sc=False,
                                     needs_layout_passes=False)


def _sc_degree_body(col_hbm, out_hbm, deg_sp, cidx_v, ones_v):
    cid = lax.axis_index("c")
    sid = lax.axis_index("s")

    def zfill(i, _):
        ones_v[pl.ds(i * 16, 16)] = jnp.zeros((16,), jnp.float32)
        return 0

    lax.fori_loop(0, RPT // 16, zfill, 0)
    pltpu.sync_copy(ones_v.at[pl.ds(0, RPT)], deg_sp.at[pl.ds(sid * RPT, RPT)])

    def fill(i, _):
        ones_v[pl.ds(i * 16, 16)] = jnp.ones((16,), jnp.float32)
        return 0

    lax.fori_loop(0, CHUNK_D // 16, fill, 0)
    plsc.subcore_barrier()

    epw = E // NW
    base = (cid * NS + sid) * epw

    def body(k, _):
        pltpu.sync_copy(col_hbm.at[pl.ds(base + k * CHUNK_D, CHUNK_D)], cidx_v)
        pltpu.sync_copy(ones_v, deg_sp.at[cidx_v], add=True)
        return 0

    lax.fori_loop(0, epw // CHUNK_D, body, 0)
    plsc.subcore_barrier()
    pltpu.sync_copy(deg_sp.at[pl.ds(sid * RPT, RPT)],
                    out_hbm.at[cid, pl.ds(sid * RPT, RPT)])


def _sc_degree(col):
    k = functools.partial(
        pl.kernel,
        out_type=jax.ShapeDtypeStruct((NC, NPAD), jnp.float32),
        mesh=_mesh,
        compiler_params=_sc_params,
        scratch_types=[
            pltpu.VMEM_SHARED((NPAD,), jnp.float32),
            pltpu.VMEM((CHUNK_D,), jnp.int32),
            pltpu.VMEM((CHUNK_D,), jnp.float32),
        ],
    )(_sc_degree_body)
    return k(col)


def _sc_edge_body(y_hbm, row2d_hbm, col2d_hbm, out_hbm,
                  y_sp, acc_sp, ridx_all, cidx_all,
                  rows_a, rows_b, sem_g0, sem_g1):
    cid = lax.axis_index("c")
    sid = lax.axis_index("s")
    fbase = cid * FH
    r0 = sid * RPT

    def zfill(i, _):
        rows_a[i // 2, pl.ds((i % 2) * 16, 16)] = jnp.zeros((16,), jnp.float32)
        return 0

    lax.fori_loop(0, CK * 2, zfill, 0)
    pltpu.sync_copy(rows_a, acc_sp.at[pl.ds(r0, CK)])
    pltpu.sync_copy(rows_a.at[pl.ds(0, RPT - CK)],
                    acc_sp.at[pl.ds(r0 + CK, RPT - CK)])
    pltpu.sync_copy(y_hbm.at[pl.ds(r0, RPT), pl.ds(fbase, FH)],
                    y_sp.at[pl.ds(r0, RPT)])
    pltpu.sync_copy(row2d_hbm.at[pl.ds(sid * CPT_P, CPT_P)], ridx_all)
    pltpu.sync_copy(col2d_hbm.at[pl.ds(sid * CPT_P, CPT_P)], cidx_all)
    plsc.subcore_barrier()

    pltpu.async_copy(y_sp.at[ridx_all.at[0]], rows_a, sem_g0)

    def body(j, _):
        k0 = 2 * j
        pltpu.make_async_copy(y_sp.at[ridx_all.at[k0]], rows_a, sem_g0).wait()
        pltpu.async_copy(y_sp.at[ridx_all.at[k0 + 1]], rows_b, sem_g1)
        pltpu.sync_copy(rows_a, acc_sp.at[cidx_all.at[k0]], add=True)
        pltpu.make_async_copy(y_sp.at[ridx_all.at[k0 + 1]], rows_b,
                              sem_g1).wait()

        @pl.when(k0 + 2 < CPT_P)
        def _():
            pltpu.async_copy(y_sp.at[ridx_all.at[k0 + 2]], rows_a, sem_g0)

        pltpu.sync_copy(rows_b, acc_sp.at[cidx_all.at[k0 + 1]], add=True)
        return 0

    lax.fori_loop(0, CPT_P // 2, body, 0)
    plsc.subcore_barrier()
    pltpu.sync_copy(acc_sp.at[pl.ds(r0, RPT)],
                    out_hbm.at[pl.ds(r0, RPT), pl.ds(fbase, FH)])


def _sc_edge_pass(y_pad, row2d, col2d):
    k = functools.partial(
        pl.kernel,
        out_type=jax.ShapeDtypeStruct((NPAD, H), jnp.float32),
        mesh=_mesh,
        compiler_params=_sc_params,
        scratch_types=[
            pltpu.VMEM_SHARED((NPAD, FH), jnp.float32),
            pltpu.VMEM_SHARED((NPAD, FH), jnp.float32),
            pltpu.VMEM((CPT_P, CK), jnp.int32),
            pltpu.VMEM((CPT_P, CK), jnp.int32),
            pltpu.VMEM((CK, FH), jnp.float32),
            pltpu.VMEM((CK, FH), jnp.float32),
            pltpu.SemaphoreType.DMA,
            pltpu.SemaphoreType.DMA,
        ],
    )(_sc_edge_body)
    return k(y_pad, row2d, col2d)


def _sc_edge_final_body(y_hbm, row2d_hbm, col2d_hbm, dinv_hbm, b2_hbm,
                        rep_out,
                        y_sp, acc_sp, ridx_all, cidx_all,
                        rows_a, rows_b, dinv_v, b2h, sem_g0, sem_g1):
    cid = lax.axis_index("c")
    sid = lax.axis_index("s")
    fbase = cid * FH
    r0 = sid * RPT

    def zfill(i, _):
        rows_a[i // 2, pl.ds((i % 2) * 16, 16)] = jnp.zeros((16,), jnp.float32)
        return 0

    lax.fori_loop(0, CK * 2, zfill, 0)
    pltpu.sync_copy(rows_a, acc_sp.at[pl.ds(r0, CK)])
    pltpu.sync_copy(rows_a.at[pl.ds(0, RPT - CK)],
                    acc_sp.at[pl.ds(r0 + CK, RPT - CK)])
    pltpu.sync_copy(y_hbm.at[pl.ds(r0, RPT), pl.ds(fbase, FH)],
                    y_sp.at[pl.ds(r0, RPT)])
    pltpu.sync_copy(row2d_hbm.at[pl.ds(sid * CPT_P, CPT_P)], ridx_all)
    pltpu.sync_copy(col2d_hbm.at[pl.ds(sid * CPT_P, CPT_P)], cidx_all)
    pltpu.sync_copy(dinv_hbm.at[pl.ds(r0, RPT)], dinv_v.at[pl.ds(0, RPT)])
    pltpu.sync_copy(b2_hbm.at[pl.ds(fbase, FH)], b2h)
    plsc.subcore_barrier()

    pltpu.async_copy(y_sp.at[ridx_all.at[0]], rows_a, sem_g0)

    def body(j, _):
        k0 = 2 * j
        pltpu.make_async_copy(y_sp.at[ridx_all.at[k0]], rows_a, sem_g0).wait()
        pltpu.async_copy(y_sp.at[ridx_all.at[k0 + 1]], rows_b, sem_g1)
        pltpu.sync_copy(rows_a, acc_sp.at[cidx_all.at[k0]], add=True)
        pltpu.make_async_copy(y_sp.at[ridx_all.at[k0 + 1]], rows_b,
                              sem_g1).wait()

        @pl.when(k0 + 2 < CPT_P)
        def _():
            pltpu.async_copy(y_sp.at[ridx_all.at[k0 + 2]], rows_a, sem_g0)

        pltpu.sync_copy(rows_b, acc_sp.at[cidx_all.at[k0 + 1]], add=True)
        return 0

    lax.fori_loop(0, CPT_P // 2, body, 0)
    plsc.subcore_barrier()

    b2v0 = b2h[pl.ds(0, 16)]
    b2v1 = b2h[pl.ds(16, 16)]

    def span(off, length):
        pltpu.sync_copy(acc_sp.at[pl.ds(r0 + off, length)],
                        rows_a.at[pl.ds(0, length)])
        pltpu.sync_copy(y_sp.at[pl.ds(r0 + off, length)],
                        rows_b.at[pl.ds(0, length)])

        def rw(r, _):
            dv = dinv_v[pl.ds(off + r, 16)][0]
            v0 = rows_a[r, pl.ds(0, 16)] + rows_b[r, pl.ds(0, 16)]
            v1 = rows_a[r, pl.ds(16, 16)] + rows_b[r, pl.ds(16, 16)]
            rows_a[r, pl.ds(0, 16)] = v0 * dv + b2v0
            rows_a[r, pl.ds(16, 16)] = v1 * dv + b2v1
            return 0

        lax.fori_loop(0, length, rw, 0)
        pltpu.sync_copy(rows_a.at[pl.ds(0, length)],
                        rep_out.at[pl.ds(r0 + off, length), pl.ds(fbase, FH)])

    span(0, CK)

    @pl.when(sid < NS - 1)
    def _():
        span(CK, RPT - CK)


def _sc_edge_final(y_pad, row2d, col2d, dinv1d, b2):
    k = functools.partial(
        pl.kernel,
        out_type=jax.ShapeDtypeStruct((N, H), jnp.float32),
        mesh=_mesh,
        compiler_params=_sc_params,
        scratch_types=[
            pltpu.VMEM_SHARED((NPAD, FH), jnp.float32),
            pltpu.VMEM_SHARED((NPAD, FH), jnp.float32),
            pltpu.VMEM((CPT_P, CK), jnp.int32),
            pltpu.VMEM((CPT_P, CK), jnp.int32),
            pltpu.VMEM((CK, FH), jnp.float32),
            pltpu.VMEM((CK, FH), jnp.float32),
            pltpu.VMEM((RPT + 16,), jnp.float32),
            pltpu.VMEM((FH,), jnp.float32),
            pltpu.SemaphoreType.DMA,
            pltpu.SemaphoreType.DMA,
        ],
    )(_sc_edge_final_body)
    return k(y_pad, row2d, col2d, dinv1d, b2)


def _sc_score_body(rep_hbm, pr_hbm, pc_hbm, nr_hbm, nc_hbm, out_hbm,
                   rep_sp, pr_idx, pc_idx, nr_idx, nc_idx,
                   ar0, br0, ar1, br1, d0, d1,
                   sem_g0, sem_g1, sem_w0, sem_w1):
    cid = lax.axis_index("c")
    sid = lax.axis_index("s")
    fbase = cid * FH
    r0 = sid * RPT

    rs0 = sid * (N // NS)
    pltpu.sync_copy(rep_hbm.at[pl.ds(rs0, N // NS), pl.ds(fbase, FH)],
                    rep_sp.at[pl.ds(rs0, N // NS)])
    pltpu.sync_copy(pr_hbm.at[pl.ds(sid * CPT_P, CPT_P)], pr_idx)
    pltpu.sync_copy(pc_hbm.at[pl.ds(sid * CPT_P, CPT_P)], pc_idx)
    pltpu.sync_copy(nr_hbm.at[pl.ds(sid * CPT_N, CPT_N)], nr_idx)
    pltpu.sync_copy(nc_hbm.at[pl.ds(sid * CPT_N, CPT_N)], nc_idx)
    plsc.subcore_barrier()

    lane = lax.iota(jnp.int32, 16)

    def compute(arows, brows, dbuf):
        def grp(g, _):
            rows0 = g * 16 + lane
            accs = [jnp.zeros((16,), jnp.float32) for _ in range(4)]
            for f in range(FH):
                colsf = jnp.bitwise_and(f + lane, FH - 1)
                a = plsc.load_gather(arows, [rows0, colsf])
                b = plsc.load_gather(brows, [rows0, colsf])
                accs[f % 4] = accs[f % 4] + a * b
            dbuf[pl.ds(g * 16, 16)] = (accs[0] + accs[1]) + (accs[2] + accs[3])
            return 0

        lax.fori_loop(0, CK // 16, grp, 0)

    def run(cpt, ridx, cidx, obase):
        def gather(t, ar, br, sem):
            pltpu.async_copy(rep_sp.at[ridx.at[t]], ar, sem)
            pltpu.async_copy(rep_sp.at[cidx.at[t]], br, sem)

        def drain(t, ar, br, sem):
            pltpu.make_async_copy(rep_sp.at[ridx.at[t]], ar, sem).wait()
            pltpu.make_async_copy(rep_sp.at[cidx.at[t]], br, sem).wait()

        gather(0, ar0, br0, sem_g0)

        def body(j, _):
            k0 = 2 * j
            drain(k0, ar0, br0, sem_g0)
            gather(k0 + 1, ar1, br1, sem_g1)

            @pl.when(j > 0)
            def _():
                pltpu.make_async_copy(
                    d0, out_hbm.at[cid, pl.ds(0, CK)], sem_w0).wait()

            compute(ar0, br0, d0)
            off0 = obase + (sid * cpt + k0) * CK
            pltpu.async_copy(d0, out_hbm.at[cid, pl.ds(off0, CK)], sem_w0)

            drain(k0 + 1, ar1, br1, sem_g1)

            @pl.when(k0 + 2 < cpt)
            def _():
                gather(k0 + 2, ar0, br0, sem_g0)

            @pl.when(j > 0)
            def _():
                pltpu.make_async_copy(
                    d1, out_hbm.at[cid, pl.ds(0, CK)], sem_w1).wait()

            compute(ar1, br1, d1)
            off1 = off0 + CK
            pltpu.async_copy(d1, out_hbm.at[cid, pl.ds(off1, CK)], sem_w1)
            return 0

        lax.fori_loop(0, cpt // 2, body, 0)
        pltpu.make_async_copy(d0, out_hbm.at[cid, pl.ds(0, CK)], sem_w0).wait()
        pltpu.make_async_copy(d1, out_hbm.at[cid, pl.ds(0, CK)], sem_w1).wait()

    run(CPT_P, pr_idx, pc_idx, 0)
    run(CPT_N, nr_idx, nc_idx, E)


def _sc_score(rep_pad, pr2d, pc2d, nr2d, nc2d):
    k = functools.partial(
        pl.kernel,
        out_type=jax.ShapeDtypeStruct((NC, ETOT), jnp.float32),
        mesh=_mesh,
        compiler_params=_sc_params_nl,
        scratch_types=[
            pltpu.VMEM_SHARED((N, FH), jnp.float32),
            pltpu.VMEM((CPT_P, CK), jnp.int32),
            pltpu.VMEM((CPT_P, CK), jnp.int32),
            pltpu.VMEM((CPT_N, CK), jnp.int32),
            pltpu.VMEM((CPT_N, CK), jnp.int32),
            pltpu.VMEM((CK, FH), jnp.float32),
            pltpu.VMEM((CK, FH), jnp.float32),
            pltpu.VMEM((CK, FH), jnp.float32),
            pltpu.VMEM((CK, FH), jnp.float32),
            pltpu.VMEM((CK,), jnp.float32),
            pltpu.VMEM((CK,), jnp.float32),
            pltpu.SemaphoreType.DMA,
            pltpu.SemaphoreType.DMA,
            pltpu.SemaphoreType.DMA,
            pltpu.SemaphoreType.DMA,
        ],
    )(_sc_score_body)
    return k(rep_pad, pr2d, pc2d, nr2d, nc2d)


def _prep_body(x_ref, w_ref, degt_ref, dinv_ref, y_ref):
    d = degt_ref[...]
    deg = d[:, 0:1] + d[:, 1:2] + 1.0
    dinv = lax.rsqrt(deg)
    dinv_ref[...] = dinv
    xw = jnp.dot(x_ref[...], w_ref[...], preferred_element_type=jnp.float32)
    y_ref[0:N, :] = dinv[0:N] * xw
    y_ref[N:NPAD, :] = jnp.zeros((NPAD - N, H), jnp.float32)


def _tc_prep(features, W1, degt):
    return pl.pallas_call(
        _prep_body,
        out_shape=[jax.ShapeDtypeStruct((NPAD, 1), jnp.float32),
                   jax.ShapeDtypeStruct((NPAD, H), jnp.float32)],
    )(features, W1, degt)


def _mid_body(acc_ref, y_ref, dinv_ref, b1_ref, w2_ref, o_ref):
    dinv = dinv_ref[...]
    s = acc_ref[...] + y_ref[...]
    h = jnp.maximum(dinv * s + b1_ref[...], 0.0)
    xw2 = jnp.dot(h, w2_ref[...], preferred_element_type=jnp.float32)
    y2 = dinv * xw2
    o_ref[0:N, :] = y2[0:N]
    o_ref[N:NPAD, :] = jnp.zeros((NPAD - N, H), jnp.float32)


def _tc_mid(acc1, y1p, dinvp, b1, W2):
    return pl.pallas_call(
        _mid_body,
        out_shape=jax.ShapeDtypeStruct((NPAD, H), jnp.float32),
    )(acc1, y1p, dinvp, b1, W2)


def _combine_body(dots_ref, pr_ref, pc_ref, nr_ref, nc_ref, o_ref):
    dp = dots_ref[0] + dots_ref[1]
    pos_d = dp[0:E // 128]
    neg_d = dp[E // 128:ETOT // 128]
    mp = (pr_ref[...] < pc_ref[...]).astype(jnp.float32)
    mn = (nr_ref[...] < nc_ref[...]).astype(jnp.float32)
    t = pos_d - 1.0
    s_pos = jnp.sum(mp * t * t)
    s_neg = jnp.sum(mn * neg_d * neg_d)
    denom = jnp.sum(mp) + jnp.sum(mn)
    rec = (s_neg + s_pos) * jnp.float32(N) / denom
    o_ref[...] = jnp.broadcast_to(rec, (1, 1))


def _tc_combine(dots3d, pr, pc, nr, nc):
    return pl.pallas_call(
        _combine_body,
        out_shape=jax.ShapeDtypeStruct((1, 1), jnp.float32),
    )(dots3d, pr, pc, nr, nc)


def kernel(features, edge_index, neg_edge_index, W1, b1, W2, b2):
    assert features.shape == (N, F_IN)
    assert edge_index.shape == (2, E)
    assert neg_edge_index.shape == (2, NNEG)

    row = edge_index[0]
    col = edge_index[1]
    nr = neg_edge_index[0]
    nc = neg_edge_index[1]
    zpad = jnp.zeros((NNEG_PAD - NNEG,), jnp.int32)
    nr_p = jnp.concatenate([nr, zpad])
    nc_p = jnp.concatenate([nc, zpad])
    row2d = row.reshape(NCH_P, CK)
    col2d = col.reshape(NCH_P, CK)
    nr2d = nr_p.reshape(NCH_N, CK)
    nc2d = nc_p.reshape(NCH_N, CK)

    degp = _sc_degree(col)
    degt = jnp.transpose(degp)
    dinvp, y1p = _tc_prep(features, W1, degt)
    acc1 = _sc_edge_pass(y1p, row2d, col2d)
    y2p = _tc_mid(acc1, y1p, dinvp, b1, W2)
    rep = _sc_edge_final(y2p, row2d, col2d, dinvp.reshape(NPAD), b2)
    dots = _sc_score(rep, row2d, col2d, nr2d, nc2d)
    dots3d = dots.reshape(NC, ETOT // 128, 128)
    rec_loss = _tc_combine(dots3d,
                           row.reshape(E // 128, 128),
                           col.reshape(E // 128, 128),
                           nr_p.reshape(NNEG_PAD // 128, 128),
                           nc_p.reshape(NNEG_PAD // 128, 128))[0, 0]
    return (rep, rec_loss)
